# Initial kernel scaffold; baseline (speedup 1.0000x reference)
import jax
import jax.numpy as jnp
from jax.experimental import pallas as pl


def kernel(x, edge_index, W, b):
    raise NotImplementedError("write your pallas kernel here")


# speedup vs baseline: 2.2474x; 2.2474x over previous
import jax
import jax.numpy as jnp
from jax import lax
from jax.experimental import pallas as pl
from jax.experimental.pallas import tpu as pltpu
from jax.experimental.pallas import tpu_sc as plsc

N = 10000
D = 256
E = 160000
NP = 10240
EP = 163840
NC, NS, LANES = 2, 16, 16
SENT = 20000
NW = NC * NS
WIN = NP // NW
CHN = EP // 1024
LISTN = 160
BIG = 1 << 30

_SC_MESH = plsc.VectorSubcoreMesh(
    core_axis_name="c", subcore_axis_name="s", num_cores=NC, num_subcores=NS)


def _bcast(vec, idx):
    return vec.at[jnp.full((LANES,), idx, jnp.int32)].get(
        mode="promise_in_bounds")


def _shuffle(vec, perm):
    return vec.at[perm].get(mode="promise_in_bounds")


def _lane_sum(vec):
    li = lax.iota(jnp.int32, LANES)
    t = vec
    for sh in (1, 2, 4, 8):
        t = t + _shuffle(t, li ^ sh)
    return t


def _lane_min(vec):
    li = lax.iota(jnp.int32, LANES)
    t = vec
    for sh in (1, 2, 4, 8):
        t = jnp.minimum(t, _shuffle(t, li ^ sh))
    return t


def _deg_body(dst_hbm, degb_hbm, didx_v, acc_v, ob_v):
    c = lax.axis_index("c")
    s = lax.axis_index("s")
    lo = (c * NS + s) * WIN
    li = lax.iota(jnp.int32, LANES)
    zero16 = jnp.zeros((LANES,), jnp.float32)

    def zacc(i, _):
        acc_v[pl.ds(i * LANES, LANES)] = zero16
        return 0
    lax.fori_loop(0, WIN // LANES, zacc, 0)

    def chunk(ch, _):
        pltpu.sync_copy(dst_hbm.at[ch], didx_v)

        def sub(i, _):
            vd = didx_v[i // 8, pl.ds((i % 8) * LANES, LANES)]
            m = (vd >= lo) & (vd < lo + WIN)
            n = _lane_sum(jnp.where(m, 1, 0))[0]

            @pl.when(n > 0)
            def _():
                enc0 = jnp.where(m, (vd - lo) * LANES + li, BIG)

                def one(t, enc):
                    mn = _lane_min(enc)
                    dle = mn[0] // LANES
                    base = (dle // LANES) * LANES
                    oh = jnp.where(li == jnp.full((LANES,), dle % LANES,
                                                  jnp.int32), 1.0, 0.0)
                    acc_v[pl.ds(base, LANES)] = acc_v[pl.ds(base, LANES)] + oh
                    return jnp.where(enc == mn, BIG, enc)
                lax.fori_loop(0, n, one, enc0)
            return 0
        lax.fori_loop(0, 64, sub, 0)
        return 0
    lax.fori_loop(0, CHN, chunk, 0)

    def obuild(g, _):
        def orow(r, _):
            rr = g * 32 + r
            cv = acc_v[pl.ds((rr // LANES) * LANES, LANES)]
            sel = jnp.where(li == jnp.full((LANES,), rr % LANES, jnp.int32),
                            cv, 0.0)
            bc = _lane_sum(sel)

            def ocol(k, _):
                ob_v[r, pl.ds(k * LANES, LANES)] = bc
                return 0
            lax.fori_loop(0, D // LANES, ocol, 0)
            return 0
        lax.fori_loop(0, 32, orow, 0)
        pltpu.sync_copy(ob_v, degb_hbm.at[pl.ds(lo + g * 32, 32)])
        return 0
    lax.fori_loop(0, WIN // 32, obuild, 0)


_deg_call = pl.kernel(
    _deg_body,
    out_type=jax.ShapeDtypeStruct((NP, D), jnp.float32),
    mesh=_SC_MESH,
    scratch_types=[
        pltpu.VMEM((8, 128), jnp.int32),
        pltpu.VMEM((WIN,), jnp.float32),
        pltpu.VMEM((32, D), jnp.float32),
    ],
)


def _mm_body(x_ref, w_ref, degb_ref, hp_ref, disb_ref):
    deg = degb_ref[:, 0:1] + 1.0
    dis = lax.rsqrt(deg)
    h = jnp.dot(x_ref[...], w_ref[...], preferred_element_type=jnp.float32)
    hp_ref[...] = h * dis
    disb_ref[...] = jnp.broadcast_to(dis, (256, 256))


_mm_call = pl.pallas_call(
    _mm_body,
    grid=(NP // 256,),
    in_specs=[
        pl.BlockSpec((256, D), lambda i: (i, 0)),
        pl.BlockSpec((D, D), lambda i: (0, 0)),
        pl.BlockSpec((256, D), lambda i: (i, 0)),
    ],
    out_specs=[
        pl.BlockSpec((256, D), lambda i: (i, 0)),
        pl.BlockSpec((256, D), lambda i: (i, 0)),
    ],
    out_shape=[
        jax.ShapeDtypeStruct((NP, D), jnp.float32),
        jax.ShapeDtypeStruct((NP, D), jnp.float32),
    ],
)


def _scat_body(hp_hbm, src_hbm, dst_hbm, agg_hbm,
               sidx_v, didx_v, slist_v, dlist_v, rows_v, acc_v, sem):
    c = lax.axis_index("c")
    s = lax.axis_index("s")
    lo = (c * NS + s) * WIN
    li = lax.iota(jnp.int32, LANES)
    zero16 = jnp.zeros((LANES,), jnp.float32)

    def zacc(i, _):
        acc_v[i // (D // LANES),
              pl.ds((i % (D // LANES)) * LANES, LANES)] = zero16
        return 0
    lax.fori_loop(0, WIN * (D // LANES), zacc, 0)

    def zlist(i, _):
        slist_v[pl.ds(i * LANES, LANES)] = jnp.zeros((LANES,), jnp.int32)
        return 0
    lax.fori_loop(0, LISTN // LANES, zlist, 0)

    def accum(cnt):
        pltpu.async_copy(hp_hbm.at[slist_v.at[pl.ds(0, 128)]], rows_v,
                         sem).wait()

        def upd(e, _):
            dle = dlist_v[pl.ds(e, LANES)][0]
            for k in range(D // LANES):
                sl = pl.ds(k * LANES, LANES)
                acc_v[dle, sl] = acc_v[dle, sl] + rows_v[e, sl]
            return 0
        lax.fori_loop(0, cnt, upd, 0)

    def chunk(ch, p):
        pltpu.sync_copy(src_hbm.at[ch], sidx_v)
        pltpu.sync_copy(dst_hbm.at[ch], didx_v)

        def sub(i, p):
            sl8 = pl.ds((i % 8) * LANES, LANES)
            vs = sidx_v[i // 8, sl8]
            vd = didx_v[i // 8, sl8]
            m = (vd >= lo) & (vd < lo + WIN)
            n = _lane_sum(jnp.where(m, 1, 0))[0]

            def matched(p):
                enc0 = jnp.where(m, (vs * 512 + (vd - lo)) * LANES + li, BIG)

                def one(t, carry):
                    pp, enc = carry
                    mn = _lane_min(enc)
                    e0 = mn[0]
                    sv = e0 // (512 * LANES)
                    dl = (e0 // LANES) % 512
                    slist_v[pl.ds(pp, LANES)] = jnp.full((LANES,), sv,
                                                         jnp.int32)
                    dlist_v[pl.ds(pp, LANES)] = jnp.full((LANES,), dl,
                                                         jnp.int32)
                    return pp + 1, jnp.where(enc == mn, BIG, enc)
                p2, _ = lax.fori_loop(0, n, one, (p, enc0))
                return p2

            p = lax.cond(n > 0, matched, lambda q: q, p)

            def fire(q):
                accum(128)
                sv = slist_v[pl.ds(128, LANES)]
                dv = dlist_v[pl.ds(128, LANES)]
                slist_v[pl.ds(0, LANES)] = sv
                dlist_v[pl.ds(0, LANES)] = dv
                return q - 128

            return lax.cond(p >= 128, fire, lambda q: q, p)
        return lax.fori_loop(0, 64, sub, p)
    p = lax.fori_loop(0, CHN, chunk, 0)
    accum(p)

    def drain(g, _):
        pltpu.sync_copy(acc_v.at[pl.ds(g * 32, 32)],
                        agg_hbm.at[pl.ds(lo + g * 32, 32)])
        return 0
    lax.fori_loop(0, WIN // 32, drain, 0)


_scat_call = pl.kernel(
    _scat_body,
    out_type=jax.ShapeDtypeStruct((NP, D), jnp.float32),
    mesh=_SC_MESH,
    scratch_types=[
        pltpu.VMEM((8, 128), jnp.int32),
        pltpu.VMEM((8, 128), jnp.int32),
        pltpu.VMEM((LISTN,), jnp.int32),
        pltpu.VMEM((LISTN,), jnp.int32),
        pltpu.VMEM((128, D), jnp.float32),
        pltpu.VMEM((WIN, D), jnp.float32),
        pltpu.SemaphoreType.DMA,
    ],
)


def _fin_body(agg_ref, hp_ref, disb_ref, b_ref, out_ref):
    out_ref[...] = (agg_ref[...] + hp_ref[...]) * disb_ref[...] + b_ref[...]


_fin_call = pl.pallas_call(
    _fin_body,
    grid=(NP // 256,),
    in_specs=[
        pl.BlockSpec((256, D), lambda i: (i, 0)),
        pl.BlockSpec((256, D), lambda i: (i, 0)),
        pl.BlockSpec((256, D), lambda i: (i, 0)),
        pl.BlockSpec((D,), lambda i: (0,)),
    ],
    out_specs=pl.BlockSpec((256, D), lambda i: (i, 0)),
    out_shape=jax.ShapeDtypeStruct((NP, D), jnp.float32),
)


def kernel(x, edge_index, W, b):
    src = edge_index[0].astype(jnp.int32)
    dst = edge_index[1].astype(jnp.int32)
    src_p = jnp.concatenate([src, jnp.zeros((EP - E,), jnp.int32)])
    dst_p = jnp.concatenate([dst, jnp.full((EP - E,), SENT, jnp.int32)])
    x_p = jnp.concatenate([x, jnp.zeros((NP - N, D), x.dtype)])

    degb = _deg_call(dst_p.reshape(CHN, 8, 128))
    hp, disb = _mm_call(x_p, W, degb)
    agg = _scat_call(hp,
                     src_p.reshape(CHN, 8, 128),
                     dst_p.reshape(CHN, 8, 128))
    out = _fin_call(agg, hp, disb, b)
    return out[:N], edge_index


# speedup vs baseline: 3.0539x; 1.3589x over previous
import jax
import jax.numpy as jnp
from jax import lax
from jax.experimental import pallas as pl
from jax.experimental.pallas import tpu as pltpu
from jax.experimental.pallas import tpu_sc as plsc

N = 10000
D = 256
E = 160000
NP = 10240
EP = 163840
NC, NS, LANES = 2, 16, 16
SENT = 20000
NW = NC * NS
WIN = NP // NW
CHN = EP // 1024
LISTN = 208
BIG = 1 << 30

_SC_MESH = plsc.VectorSubcoreMesh(
    core_axis_name="c", subcore_axis_name="s", num_cores=NC, num_subcores=NS)


def _bcast(vec, idx):
    return vec.at[jnp.full((LANES,), idx, jnp.int32)].get(
        mode="promise_in_bounds")


def _shuffle(vec, perm):
    return vec.at[perm].get(mode="promise_in_bounds")


def _lane_sum(vec):
    li = lax.iota(jnp.int32, LANES)
    t = vec
    for sh in (1, 2, 4, 8):
        t = t + _shuffle(t, li ^ sh)
    return t


def _lane_min(vec):
    li = lax.iota(jnp.int32, LANES)
    t = vec
    for sh in (1, 2, 4, 8):
        t = jnp.minimum(t, _shuffle(t, li ^ sh))
    return t


def _deg_body(dst_hbm, degb_hbm, didx_v, acc_v, ob_v):
    c = lax.axis_index("c")
    s = lax.axis_index("s")
    lo = (c * NS + s) * WIN
    li = lax.iota(jnp.int32, LANES)
    zero16 = jnp.zeros((LANES,), jnp.float32)

    def zacc(i, _):
        acc_v[pl.ds(i * LANES, LANES)] = zero16
        return 0
    lax.fori_loop(0, WIN // LANES, zacc, 0)

    def chunk(ch, _):
        pltpu.sync_copy(dst_hbm.at[ch], didx_v)

        def sub(g, _):
            vds, ms = [], []
            for q in range(4):
                vd = didx_v[g // 2, pl.ds((g % 2) * 64 + q * LANES, LANES)]
                vl = vd - lo
                vds.append(vl)
                ms.append((vl >= 0) & (vl < WIN))
            tot = (jnp.where(ms[0], 1, 0) + jnp.where(ms[1], 1, 0)
                   + jnp.where(ms[2], 1, 0) + jnp.where(ms[3], 1, 0))
            n = _lane_sum(tot)[0]

            @pl.when(n > 0)
            def _():
                encs = [jnp.where(ms[q], (vds[q] * 4 + q) * LANES + li, BIG)
                        for q in range(4)]

                def one(t, encs):
                    e0, e1, e2, e3 = encs
                    mn = _lane_min(jnp.minimum(jnp.minimum(e0, e1),
                                               jnp.minimum(e2, e3)))
                    dle = mn[0] // (4 * LANES)
                    base = (dle // LANES) * LANES
                    oh = jnp.where(li == jnp.full((LANES,), dle % LANES,
                                                  jnp.int32), 1.0, 0.0)
                    acc_v[pl.ds(base, LANES)] = acc_v[pl.ds(base, LANES)] + oh
                    return tuple(jnp.where(e == mn, BIG, e)
                                 for e in (e0, e1, e2, e3))
                lax.fori_loop(0, n, one, tuple(encs))
            return 0
        lax.fori_loop(0, 16, sub, 0)
        return 0
    lax.fori_loop(0, CHN, chunk, 0)

    def obuild(g, _):
        def orow(r, _):
            rr = g * 32 + r
            cv = acc_v[pl.ds((rr // LANES) * LANES, LANES)]
            sel = jnp.where(li == jnp.full((LANES,), rr % LANES, jnp.int32),
                            cv, 0.0)
            bc = _lane_sum(sel)

            def ocol(k, _):
                ob_v[r, pl.ds(k * LANES, LANES)] = bc
                return 0
            lax.fori_loop(0, D // LANES, ocol, 0)
            return 0
        lax.fori_loop(0, 32, orow, 0)
        pltpu.sync_copy(ob_v, degb_hbm.at[pl.ds(lo + g * 32, 32)])
        return 0
    lax.fori_loop(0, WIN // 32, obuild, 0)


_deg_call = pl.kernel(
    _deg_body,
    out_type=jax.ShapeDtypeStruct((NP, D), jnp.float32),
    mesh=_SC_MESH,
    scratch_types=[
        pltpu.VMEM((8, 128), jnp.int32),
        pltpu.VMEM((WIN,), jnp.float32),
        pltpu.VMEM((32, D), jnp.float32),
    ],
)


def _mm_body(x_ref, w_ref, degb_ref, hp_ref, disb_ref):
    deg = degb_ref[:, 0:1] + 1.0
    dis = lax.rsqrt(deg)
    h = jnp.dot(x_ref[...], w_ref[...], preferred_element_type=jnp.float32)
    hp_ref[...] = h * dis
    disb_ref[...] = jnp.broadcast_to(dis, (256, 256))


_mm_call = pl.pallas_call(
    _mm_body,
    grid=(NP // 256,),
    in_specs=[
        pl.BlockSpec((256, D), lambda i: (i, 0)),
        pl.BlockSpec((D, D), lambda i: (0, 0)),
        pl.BlockSpec((256, D), lambda i: (i, 0)),
    ],
    out_specs=[
        pl.BlockSpec((256, D), lambda i: (i, 0)),
        pl.BlockSpec((256, D), lambda i: (i, 0)),
    ],
    out_shape=[
        jax.ShapeDtypeStruct((NP, D), jnp.float32),
        jax.ShapeDtypeStruct((NP, D), jnp.float32),
    ],
)


def _scat_body(hp_hbm, src_hbm, dst_hbm, agg_hbm,
               sidx_v, didx_v, slist_v, dlist_v, rows_v, acc_v, sem):
    c = lax.axis_index("c")
    s = lax.axis_index("s")
    lo = (c * NS + s) * WIN
    li = lax.iota(jnp.int32, LANES)
    zero16 = jnp.zeros((LANES,), jnp.float32)

    def zacc(i, _):
        acc_v[i // (D // LANES),
              pl.ds((i % (D // LANES)) * LANES, LANES)] = zero16
        return 0
    lax.fori_loop(0, WIN * (D // LANES), zacc, 0)

    def zlist(i, _):
        slist_v[pl.ds(i * LANES, LANES)] = jnp.zeros((LANES,), jnp.int32)
        return 0
    lax.fori_loop(0, LISTN // LANES, zlist, 0)

    def accum(cnt):
        pltpu.async_copy(hp_hbm.at[slist_v.at[pl.ds(0, 128)]], rows_v,
                         sem).wait()

        def upd(e, _):
            dle = dlist_v[pl.ds(e, LANES)][0]
            for k in range(D // LANES):
                sl = pl.ds(k * LANES, LANES)
                acc_v[dle, sl] = acc_v[dle, sl] + rows_v[e, sl]
            return 0
        lax.fori_loop(0, cnt, upd, 0)

    def chunk(ch, p):
        pltpu.sync_copy(src_hbm.at[ch], sidx_v)
        pltpu.sync_copy(dst_hbm.at[ch], didx_v)

        def sub(g, p):
            vss, vds, ms = [], [], []
            for q in range(4):
                sl = pl.ds((g % 2) * 64 + q * LANES, LANES)
                vds.append(didx_v[g // 2, sl] - lo)
                vss.append(sidx_v[g // 2, sl])
                ms.append((vds[q] >= 0) & (vds[q] < WIN))
            tot = (jnp.where(ms[0], 1, 0) + jnp.where(ms[1], 1, 0)
                   + jnp.where(ms[2], 1, 0) + jnp.where(ms[3], 1, 0))
            n = _lane_sum(tot)[0]

            def matched(p):
                encs = [jnp.where(ms[q],
                                  ((vss[q] * 512 + vds[q]) * 4 + q) * LANES
                                  + li, BIG)
                        for q in range(4)]

                def one(t, carry):
                    pp, e0, e1, e2, e3 = carry
                    mn = _lane_min(jnp.minimum(jnp.minimum(e0, e1),
                                               jnp.minimum(e2, e3)))
                    v0 = mn[0] // (4 * LANES)
                    sv = v0 // 512
                    dl = v0 % 512
                    slist_v[pl.ds(pp, LANES)] = jnp.full((LANES,), sv,
                                                         jnp.int32)
                    dlist_v[pl.ds(pp, LANES)] = jnp.full((LANES,), dl,
                                                         jnp.int32)
                    return (pp + 1,) + tuple(jnp.where(e == mn, BIG, e)
                                             for e in (e0, e1, e2, e3))
                out = lax.fori_loop(0, n, one, (p,) + tuple(encs))
                return out[0]

            p = lax.cond(n > 0, matched, lambda q: q, p)

            def fire(q):
                accum(128)
                for t in range(4):
                    sv = slist_v[pl.ds(128 + t * LANES, LANES)]
                    dv = dlist_v[pl.ds(128 + t * LANES, LANES)]
                    slist_v[pl.ds(t * LANES, LANES)] = sv
                    dlist_v[pl.ds(t * LANES, LANES)] = dv
                return q - 128

            return lax.cond(p >= 128, fire, lambda q: q, p)
        return lax.fori_loop(0, 16, sub, p)
    p = lax.fori_loop(0, CHN, chunk, 0)
    accum(p)

    def drain(g, _):
        pltpu.sync_copy(acc_v.at[pl.ds(g * 32, 32)],
                        agg_hbm.at[pl.ds(lo + g * 32, 32)])
        return 0
    lax.fori_loop(0, WIN // 32, drain, 0)


_scat_call = pl.kernel(
    _scat_body,
    out_type=jax.ShapeDtypeStruct((NP, D), jnp.float32),
    mesh=_SC_MESH,
    scratch_types=[
        pltpu.VMEM((8, 128), jnp.int32),
        pltpu.VMEM((8, 128), jnp.int32),
        pltpu.VMEM((LISTN,), jnp.int32),
        pltpu.VMEM((LISTN,), jnp.int32),
        pltpu.VMEM((128, D), jnp.float32),
        pltpu.VMEM((WIN, D), jnp.float32),
        pltpu.SemaphoreType.DMA,
    ],
)


def _fin_body(agg_ref, hp_ref, disb_ref, b_ref, out_ref):
    out_ref[...] = (agg_ref[...] + hp_ref[...]) * disb_ref[...] + b_ref[...]


_fin_call = pl.pallas_call(
    _fin_body,
    grid=(NP // 256,),
    in_specs=[
        pl.BlockSpec((256, D), lambda i: (i, 0)),
        pl.BlockSpec((256, D), lambda i: (i, 0)),
        pl.BlockSpec((256, D), lambda i: (i, 0)),
        pl.BlockSpec((D,), lambda i: (0,)),
    ],
    out_specs=pl.BlockSpec((256, D), lambda i: (i, 0)),
    out_shape=jax.ShapeDtypeStruct((NP, D), jnp.float32),
)


def kernel(x, edge_index, W, b):
    src = edge_index[0].astype(jnp.int32)
    dst = edge_index[1].astype(jnp.int32)
    src_p = jnp.concatenate([src, jnp.zeros((EP - E,), jnp.int32)])
    dst_p = jnp.concatenate([dst, jnp.full((EP - E,), SENT, jnp.int32)])
    x_p = jnp.concatenate([x, jnp.zeros((NP - N, D), x.dtype)])

    degb = _deg_call(dst_p.reshape(CHN, 8, 128))
    hp, disb = _mm_call(x_p, W, degb)
    agg = _scat_call(hp,
                     src_p.reshape(CHN, 8, 128),
                     dst_p.reshape(CHN, 8, 128))
    out = _fin_call(agg, hp, disb, b)
    return out[:N], edge_index
